# Initial kernel scaffold; baseline (speedup 1.0000x reference)
import jax
import jax.numpy as jnp
from jax.experimental import pallas as pl


def kernel(X, Z, Lu_raw, mu):
    raise NotImplementedError("write your pallas kernel here")


# speedup vs baseline: 36.5158x; 36.5158x over previous
import jax
import jax.numpy as jnp
from jax import lax
from jax.experimental import pallas as pl

N = 16384
M = 256
K = 8
DIM = 16
JITTER = 1e-4
LENGTHSCALE = 1.0
VARIANCE = 1.0
Q = 256

_F32 = jnp.float32


def _prep_body(z_ref, lur_ref, lu_ref, l_ref, s_ref, z2_ref):
    Z = z_ref[:]
    raw = lur_ref[:]
    r = lax.broadcasted_iota(jnp.int32, (M, M), 0)
    c = lax.broadcasted_iota(jnp.int32, (M, M), 1)
    lu = jnp.where(c < r, raw, 0.0) + jnp.where(c == r, jnp.exp(raw), 0.0)
    lu_ref[:] = lu
    s_ref[:] = lax.dot_general(lu, lu, (((1,), (1,)), ((), ())),
                               preferred_element_type=_F32)
    z2c = jnp.sum(Z * Z, axis=1, keepdims=True)
    z2r = lax.dot_general(jnp.ones((1, DIM), _F32), Z * Z,
                          (((1,), (1,)), ((), ())),
                          preferred_element_type=_F32)
    z2_ref[:] = z2r
    zz = lax.dot_general(Z, Z, (((1,), (1,)), ((), ())),
                         preferred_element_type=_F32)
    d2 = jnp.maximum(z2c + z2r - 2.0 * zz, 0.0)
    kzz = VARIANCE * jnp.exp(-0.5 * d2 / (LENGTHSCALE ** 2))
    eye = (r == c).astype(_F32)
    aw0 = kzz + JITTER * eye
    c_row = lax.broadcasted_iota(jnp.int32, (1, M), 1)
    r_col = lax.broadcasted_iota(jnp.int32, (M, 1), 0)

    def body(j, carry):
        aw, lt = carry
        ehot = (c_row == j).astype(_F32)
        rowj = lax.dot_general(ehot, aw, (((1,), (0,)), ((), ())),
                               preferred_element_type=_F32)
        piv = jnp.sum(rowj * ehot)
        d = jnp.sqrt(piv)
        u = jnp.where(c_row > j, rowj, 0.0) / d
        lrow = u + d * ehot
        lt = lt + (r_col == j).astype(_F32) * lrow
        outer = lax.dot_general(u, u, (((0,), (0,)), ((), ())),
                                preferred_element_type=_F32)
        return aw - outer, lt

    aw, lt = lax.fori_loop(0, M, body, (aw0, jnp.zeros((M, M), _F32)))
    l_ref[:] = lax.dot_general(eye, lt, (((1,), (1,)), ((), ())),
                               preferred_element_type=_F32)


def _main_body(x_ref, z_ref, z2_ref, s_ref, mu_ref, mean_ref, scale_ref):
    x = x_ref[:]
    Z = z_ref[:]
    z2 = z2_ref[:]
    x2 = jnp.sum(x * x, axis=1, keepdims=True)
    xz = lax.dot_general(x, Z, (((1,), (1,)), ((), ())),
                         preferred_element_type=_F32)
    d2 = jnp.maximum(x2 + z2 - 2.0 * xz, 0.0)
    lane = lax.broadcasted_iota(jnp.int32, (Q, M), 1)
    idxs, vals = [], []
    d2w = d2
    for _ in range(K):
        mval = jnp.min(d2w, axis=1, keepdims=True)
        cand = jnp.where(d2w == mval, lane, M)
        am = jnp.min(cand, axis=1, keepdims=True)
        idxs.append(am)
        vals.append(mval)
        d2w = jnp.where(lane == am, _F32(3.4e38), d2w)
    idx = jnp.concatenate(idxs, axis=1)
    d2k = jnp.concatenate(vals, axis=1)
    kxz = VARIANCE * jnp.exp(-0.5 * d2k / (LENGTHSCALE ** 2))

    lane3 = lax.broadcasted_iota(jnp.int32, (Q, K, M), 2)
    P = (lane3 == idx[:, :, None]).astype(_F32)
    Pr = P.reshape(Q * K, M)
    zg = lax.dot_general(Pr, Z, (((1,), (0,)), ((), ())),
                         preferred_element_type=_F32)
    mug = lax.dot_general(Pr, mu_ref[:], (((1,), (0,)), ((), ())),
                          preferred_element_type=_F32)
    sgr = lax.dot_general(Pr, s_ref[:], (((1,), (0,)), ((), ())),
                          preferred_element_type=_F32)
    zg3 = zg.reshape(Q, K, DIM)
    sg3 = sgr.reshape(Q, K, M)
    s_sub = lax.dot_general(sg3, P, (((2,), (2,)), ((0,), (0,))),
                            preferred_element_type=_F32)
    g = lax.dot_general(zg3, zg3, (((2,), (2,)), ((0,), (0,))),
                        preferred_element_type=_F32)
    z2k = jnp.sum(zg3 * zg3, axis=2)
    d2z = jnp.maximum(z2k[:, :, None] + z2k[:, None, :] - 2.0 * g, 0.0)
    ks = VARIANCE * jnp.exp(-0.5 * d2z / (LENGTHSCALE ** 2))
    r3 = lax.broadcasted_iota(jnp.int32, (Q, K, K), 1)
    c3 = lax.broadcasted_iota(jnp.int32, (Q, K, K), 2)
    eye_k = (r3 == c3).astype(_F32)
    a_sub = ks + JITTER * eye_k
    b = ks + 2.0 * JITTER * eye_k

    binv = eye_k
    aw = b
    for k in range(K):
        piv = aw[:, k:k + 1, k:k + 1]
        pr_a = aw[:, k:k + 1, :] / piv
        pr_i = binv[:, k:k + 1, :] / piv
        colf = aw[:, :, k:k + 1]
        isrow = r3[:, :, :1] == k
        f = jnp.where(isrow, 0.0, colf)
        aw = jnp.where(isrow, pr_a, aw - f * pr_a)
        binv = jnp.where(isrow, pr_i, binv - f * pr_i)

    w = jnp.sum(kxz[:, :, None] * binv, axis=1)
    u_k = jnp.sum(a_sub * w[:, None, :], axis=2)
    quad_k = jnp.sum(w * u_k, axis=1)
    u_s = jnp.sum(s_sub * w[:, None, :], axis=2)
    quad_s = jnp.sum(w * u_s, axis=1)
    mean = jnp.sum(w * mug.reshape(Q, K), axis=1)
    cov = VARIANCE - quad_k + quad_s
    scale = jnp.sqrt(jnp.clip(cov, 0.05, None))
    mean_ref[:] = mean[:, None]
    scale_ref[:] = scale[:, None]


def kernel(X, Z, Lu_raw, mu):
    lu, l_mat, s_mat, z2 = pl.pallas_call(
        _prep_body,
        out_shape=(
            jax.ShapeDtypeStruct((M, M), _F32),
            jax.ShapeDtypeStruct((M, M), _F32),
            jax.ShapeDtypeStruct((M, M), _F32),
            jax.ShapeDtypeStruct((1, M), _F32),
        ),
    )(Z, Lu_raw)

    mu2 = mu.reshape(M, 1)
    grid = (N // Q,)
    mean_c, scale_c = pl.pallas_call(
        _main_body,
        grid=grid,
        in_specs=[
            pl.BlockSpec((Q, DIM), lambda i: (i, 0)),
            pl.BlockSpec((M, DIM), lambda i: (0, 0)),
            pl.BlockSpec((1, M), lambda i: (0, 0)),
            pl.BlockSpec((M, M), lambda i: (0, 0)),
            pl.BlockSpec((M, 1), lambda i: (0, 0)),
        ],
        out_specs=[
            pl.BlockSpec((Q, 1), lambda i: (i, 0)),
            pl.BlockSpec((Q, 1), lambda i: (i, 0)),
        ],
        out_shape=(
            jax.ShapeDtypeStruct((N, 1), _F32),
            jax.ShapeDtypeStruct((N, 1), _F32),
        ),
    )(X, Z, z2, s_mat, mu2)

    mean = mean_c.reshape(1, N)
    scale = scale_c.reshape(1, N)
    return mean, scale, mu, lu, l_mat


# speedup vs baseline: 86.3799x; 2.3656x over previous
import jax
import jax.numpy as jnp
from jax import lax
from jax.experimental import pallas as pl
from jax.experimental.pallas import tpu as pltpu

N = 16384
M = 256
K = 8
DIM = 16
JITTER = 1e-4
LENGTHSCALE = 1.0
VARIANCE = 1.0
Q = 256
NB = 128

_F32 = jnp.float32


def _prep_body(z_ref, lur_ref, lu_ref, l_ref, s_ref, zt_ref, z2_ref, lt_ref):
    Z = z_ref[:]
    raw = lur_ref[:]
    r = lax.broadcasted_iota(jnp.int32, (M, M), 0)
    c = lax.broadcasted_iota(jnp.int32, (M, M), 1)
    eye = (r == c).astype(_F32)
    lu = jnp.where(c < r, raw, 0.0) + jnp.where(c == r, jnp.exp(raw), 0.0)
    lu_ref[:] = lu
    s_ref[:] = lax.dot_general(lu, lu, (((1,), (1,)), ((), ())),
                               preferred_element_type=_F32)
    zt_ref[:] = lax.dot_general(Z, eye, (((0,), (0,)), ((), ())),
                                preferred_element_type=_F32)
    z2c = jnp.sum(Z * Z, axis=1, keepdims=True)
    z2r = lax.dot_general(jnp.ones((1, DIM), _F32), Z * Z,
                          (((1,), (1,)), ((), ())),
                          preferred_element_type=_F32)
    z2_ref[:] = z2r
    zz = lax.dot_general(Z, Z, (((1,), (1,)), ((), ())),
                         preferred_element_type=_F32)
    d2 = jnp.maximum(z2c + z2r - 2.0 * zz, 0.0)
    kzz = VARIANCE * jnp.exp(-0.5 * d2 / (LENGTHSCALE ** 2))
    aw0 = kzz + JITTER * eye

    c_nb = lax.broadcasted_iota(jnp.int32, (1, NB), 1)
    c_m = lax.broadcasted_iota(jnp.int32, (1, M), 1)
    r_m = lax.broadcasted_iota(jnp.int32, (M, 1), 0)
    eye_m = eye

    def body0(j, p0):
        ehot = (c_nb == j).astype(_F32)
        colj = lax.dot_general(p0, ehot, (((1,), (1,)), ((), ())),
                               preferred_element_type=_F32)
        piv = jnp.sum(colj * (r_m == j).astype(_F32))
        d = jnp.sqrt(piv)
        u = jnp.where(r_m > j, colj, 0.0) / d
        ucol = u + d * (r_m == j).astype(_F32)
        urow = lax.dot_general(ucol, eye_m, (((0,), (0,)), ((), ())),
                               preferred_element_type=_F32)
        lt_ref[pl.ds(j, 1), :] = urow
        outer = lax.dot_general(u, urow[:, :NB], (((1,), (0,)), ((), ())),
                                preferred_element_type=_F32)
        return p0 - outer

    lax.fori_loop(0, NB, body0, aw0[:, :NB])

    g = lt_ref[0:NB, NB:M]
    a22 = aw0[NB:M, NB:M] - lax.dot_general(
        g, g, (((0,), (0,)), ((), ())), preferred_element_type=_F32)

    c_nb2 = c_nb
    eye_nb = (lax.broadcasted_iota(jnp.int32, (NB, NB), 0)
              == lax.broadcasted_iota(jnp.int32, (NB, NB), 1)).astype(_F32)

    def body1(j, p1):
        ehot = (c_nb2 == j).astype(_F32)
        rowj = lax.dot_general(ehot, p1, (((1,), (0,)), ((), ())),
                               preferred_element_type=_F32)
        piv = jnp.sum(rowj * ehot)
        d = jnp.sqrt(piv)
        u = jnp.where(c_nb2 > j, rowj, 0.0) / d
        lrow = u + d * ehot
        lt_ref[pl.ds(NB + j, 1), :] = jnp.concatenate(
            [jnp.zeros((1, NB), _F32), lrow], axis=1)
        outer = lax.dot_general(u, u, (((0,), (0,)), ((), ())),
                                preferred_element_type=_F32)
        return p1 - outer

    lax.fori_loop(0, NB, body1, a22)

    l_ref[:] = lax.dot_general(eye_m, lt_ref[:], (((1,), (1,)), ((), ())),
                               preferred_element_type=_F32)


def _main_body(x_ref, zt_ref, z2_ref, s_ref, mu_ref, mean_ref, scale_ref):
    x = x_ref[:]
    zt = zt_ref[:]
    z2 = z2_ref[:]
    x2 = jnp.sum(x * x, axis=1, keepdims=True)
    xz = lax.dot_general(x, zt, (((1,), (0,)), ((), ())),
                         preferred_element_type=_F32)
    d2 = jnp.maximum(x2 + z2 - 2.0 * xz, 0.0)
    lane = lax.broadcasted_iota(jnp.int32, (Q, M), 1)
    idxs, vals = [], []
    d2w = d2
    for _ in range(K):
        mval = jnp.min(d2w, axis=1, keepdims=True)
        cand = jnp.where(d2w == mval, lane, M)
        am = jnp.min(cand, axis=1, keepdims=True)
        idxs.append(am.astype(_F32))
        vals.append(mval)
        d2w = jnp.where(lane == am, _F32(3.4e38), d2w)
    idxf = jnp.concatenate(idxs, axis=1)
    d2k = jnp.concatenate(vals, axis=1)

    r_q = lax.broadcasted_iota(jnp.int32, (Q, Q), 0)
    c_q = lax.broadcasted_iota(jnp.int32, (Q, Q), 1)
    eye_q = (r_q == c_q).astype(_F32)
    idx_t = lax.dot_general(idxf, eye_q, (((0,), (0,)), ((), ())),
                            preferred_element_type=_F32)
    d2k_t = lax.dot_general(d2k, eye_q, (((0,), (0,)), ((), ())),
                            preferred_element_type=_F32)
    kxz_t = VARIANCE * jnp.exp(-0.5 * d2k_t / (LENGTHSCALE ** 2))
    idx_ti = idx_t.astype(jnp.int32)

    r_mq = lax.broadcasted_iota(jnp.int32, (M, Q), 0)
    ph = [(r_mq == idx_ti[j:j + 1, :]).astype(_F32) for j in range(K)]
    zg = [lax.dot_general(zt, ph[j], (((1,), (0,)), ((), ())),
                          preferred_element_type=_F32) for j in range(K)]
    z2k = [jnp.sum(zg[j] * zg[j], axis=0, keepdims=True) for j in range(K)]

    gpair = {}
    for i in range(K):
        for j in range(i, K):
            gpair[(i, j)] = jnp.sum(zg[i] * zg[j], axis=0, keepdims=True)
    rows = []
    for i in range(K):
        for j in range(K):
            gij = gpair[(i, j)] if i <= j else gpair[(j, i)]
            rows.append(z2k[i] + z2k[j] - 2.0 * gij)
    d2z = jnp.maximum(jnp.concatenate(rows, axis=0), 0.0)
    ks = VARIANCE * jnp.exp(-0.5 * d2z / (LENGTHSCALE ** 2))
    r_kk1 = lax.broadcasted_iota(jnp.int32, (K, K, 1), 0)
    c_kk1 = lax.broadcasted_iota(jnp.int32, (K, K, 1), 1)
    b = ks.reshape(K, K, Q) + 2.0 * JITTER * (r_kk1 == c_kk1).astype(_F32)

    binv = (r_kk1 == c_kk1).astype(_F32) * jnp.ones((K, K, Q), _F32)
    aw = b
    for k in range(K):
        piv = aw[k:k + 1, k:k + 1, :]
        pr_a = aw[k:k + 1, :, :] / piv
        pr_i = binv[k:k + 1, :, :] / piv
        colf = aw[:, k:k + 1, :]
        isrow = r_kk1[:, 0:1, :] == k
        f = jnp.where(isrow, 0.0, colf)
        aw = jnp.where(isrow, pr_a, aw - f * pr_a)
        binv = jnp.where(isrow, pr_i, binv - f * pr_i)

    w = jnp.sum(kxz_t[:, None, :] * binv, axis=0)

    wg = ph[0] * w[0:1, :]
    for j in range(1, K):
        wg = wg + ph[j] * w[j:j + 1, :]

    swg = lax.dot_general(s_ref[:], wg, (((1,), (0,)), ((), ())),
                          preferred_element_type=_F32)
    quad_s = jnp.sum(wg * swg, axis=0, keepdims=True)
    mean = jnp.sum(wg * mu_ref[:], axis=0, keepdims=True)
    quad_k = (jnp.sum(w * kxz_t, axis=0, keepdims=True)
              - JITTER * jnp.sum(w * w, axis=0, keepdims=True))
    cov = VARIANCE - quad_k + quad_s
    mean_ref[:] = mean
    scale_ref[:] = jnp.sqrt(jnp.clip(cov, 0.05, None))


def kernel(X, Z, Lu_raw, mu):
    lu, l_mat, s_mat, zt, z2 = pl.pallas_call(
        _prep_body,
        out_shape=(
            jax.ShapeDtypeStruct((M, M), _F32),
            jax.ShapeDtypeStruct((M, M), _F32),
            jax.ShapeDtypeStruct((M, M), _F32),
            jax.ShapeDtypeStruct((DIM, M), _F32),
            jax.ShapeDtypeStruct((1, M), _F32),
        ),
        scratch_shapes=[pltpu.VMEM((M, M), _F32)],
    )(Z, Lu_raw)

    mu2 = mu.reshape(M, 1)
    grid = (N // Q,)
    mean, scale = pl.pallas_call(
        _main_body,
        grid=grid,
        in_specs=[
            pl.BlockSpec((Q, DIM), lambda i: (i, 0)),
            pl.BlockSpec((DIM, M), lambda i: (0, 0)),
            pl.BlockSpec((1, M), lambda i: (0, 0)),
            pl.BlockSpec((M, M), lambda i: (0, 0)),
            pl.BlockSpec((M, 1), lambda i: (0, 0)),
        ],
        out_specs=[
            pl.BlockSpec((1, Q), lambda i: (0, i)),
            pl.BlockSpec((1, Q), lambda i: (0, i)),
        ],
        out_shape=(
            jax.ShapeDtypeStruct((1, N), _F32),
            jax.ShapeDtypeStruct((1, N), _F32),
        ),
    )(X, zt, z2, s_mat, mu2)

    return mean, scale, mu, lu, l_mat


# speedup vs baseline: 113.4143x; 1.3130x over previous
import jax
import jax.numpy as jnp
from jax import lax
from jax.experimental import pallas as pl
from jax.experimental.pallas import tpu as pltpu

N = 16384
M = 256
K = 8
DIM = 16
JITTER = 1e-4
LENGTHSCALE = 1.0
VARIANCE = 1.0
Q = 256
NB = 128

_F32 = jnp.float32


def _prep_body(z_ref, lur_ref, lu_ref, l_ref, s_ref, zt_ref, z2_ref,
               lt_ref, p0t_ref, p1_ref):
    Z = z_ref[:]
    raw = lur_ref[:]
    r = lax.broadcasted_iota(jnp.int32, (M, M), 0)
    c = lax.broadcasted_iota(jnp.int32, (M, M), 1)
    eye = (r == c).astype(_F32)
    lu = jnp.where(c < r, raw, 0.0) + jnp.where(c == r, jnp.exp(raw), 0.0)
    lu_ref[:] = lu
    s_ref[:] = lax.dot_general(lu, lu, (((1,), (1,)), ((), ())),
                               preferred_element_type=_F32)
    zt_ref[:] = lax.dot_general(Z, eye, (((0,), (0,)), ((), ())),
                                preferred_element_type=_F32)
    z2c = jnp.sum(Z * Z, axis=1, keepdims=True)
    z2r = lax.dot_general(jnp.ones((1, DIM), _F32), Z * Z,
                          (((1,), (1,)), ((), ())),
                          preferred_element_type=_F32)
    z2_ref[:] = z2r
    zz = lax.dot_general(Z, Z, (((1,), (1,)), ((), ())),
                         preferred_element_type=_F32)
    d2 = jnp.maximum(z2c + z2r - 2.0 * zz, 0.0)
    kzz = VARIANCE * jnp.exp(-0.5 * d2 / (LENGTHSCALE ** 2))
    aw0 = kzz + JITTER * eye

    c_nb = lax.broadcasted_iota(jnp.int32, (1, NB), 1)
    c_m = lax.broadcasted_iota(jnp.int32, (1, M), 1)
    eye_m = eye

    p0t_ref[:] = aw0[0:NB, :]

    def body0(j, carry):
        rowj = p0t_ref[pl.ds(j, 1), :]
        ehot = (c_m == j).astype(_F32)
        piv = jnp.sum(rowj * ehot)
        d = jnp.sqrt(piv)
        u = jnp.where(c_m > j, rowj, 0.0) / d
        lt_ref[pl.ds(j, 1), :] = u + d * ehot
        outer = lax.dot_general(u[:, 0:NB], u, (((0,), (0,)), ((), ())),
                                preferred_element_type=_F32)
        p0t_ref[:] = p0t_ref[:] - outer
        return carry

    lax.fori_loop(0, NB, body0, 0)

    g = lt_ref[0:NB, NB:M]
    p1_ref[:] = aw0[NB:M, NB:M] - lax.dot_general(
        g, g, (((0,), (0,)), ((), ())), preferred_element_type=_F32)

    def body1(j, carry):
        rowj = p1_ref[pl.ds(j, 1), :]
        ehot = (c_nb == j).astype(_F32)
        piv = jnp.sum(rowj * ehot)
        d = jnp.sqrt(piv)
        u = jnp.where(c_nb > j, rowj, 0.0) / d
        lt_ref[pl.ds(NB + j, 1), :] = jnp.concatenate(
            [jnp.zeros((1, NB), _F32), u + d * ehot], axis=1)
        outer = lax.dot_general(u, u, (((0,), (0,)), ((), ())),
                                preferred_element_type=_F32)
        p1_ref[:] = p1_ref[:] - outer
        return carry

    lax.fori_loop(0, NB, body1, 0)

    l_ref[:] = lax.dot_general(eye_m, lt_ref[:], (((1,), (1,)), ((), ())),
                               preferred_element_type=_F32)


def _main_body(x_ref, zt_ref, z2_ref, s_ref, mu_ref, mean_ref, scale_ref):
    x = x_ref[:]
    zt = zt_ref[:]
    z2 = z2_ref[:]
    x2 = jnp.sum(x * x, axis=1, keepdims=True)
    xz = lax.dot_general(x, zt, (((1,), (0,)), ((), ())),
                         preferred_element_type=_F32)
    d2 = jnp.maximum(x2 + z2 - 2.0 * xz, 0.0)
    lane_f = lax.broadcasted_iota(jnp.int32, (Q, M), 1).astype(_F32)
    idxs, vals = [], []
    d2w = d2
    for _ in range(K):
        mval = jnp.min(d2w, axis=1, keepdims=True)
        cand = jnp.where(d2w == mval, lane_f, _F32(1e9))
        am = jnp.min(cand, axis=1, keepdims=True)
        idxs.append(am)
        vals.append(mval)
        d2w = jnp.where(lane_f == am, _F32(3.4e38), d2w)
    idxf = jnp.concatenate(idxs, axis=1)
    d2k = jnp.concatenate(vals, axis=1)

    r_q = lax.broadcasted_iota(jnp.int32, (Q, Q), 0)
    c_q = lax.broadcasted_iota(jnp.int32, (Q, Q), 1)
    eye_q = (r_q == c_q).astype(_F32)
    idx_t = lax.dot_general(idxf, eye_q, (((0,), (0,)), ((), ())),
                            preferred_element_type=_F32)
    d2k_t = lax.dot_general(d2k, eye_q, (((0,), (0,)), ((), ())),
                            preferred_element_type=_F32)
    kxz_t = VARIANCE * jnp.exp(-0.5 * d2k_t / (LENGTHSCALE ** 2))

    r_mq = lax.broadcasted_iota(jnp.int32, (M, Q), 0).astype(_F32)
    ph = [(r_mq == idx_t[j:j + 1, :]).astype(_F32) for j in range(K)]
    zg = [lax.dot_general(zt, ph[j], (((1,), (0,)), ((), ())),
                          preferred_element_type=_F32) for j in range(K)]
    z2k = [jnp.sum(zg[j] * zg[j], axis=0, keepdims=True) for j in range(K)]

    gpair = {}
    for i in range(K):
        for j in range(i, K):
            gpair[(i, j)] = jnp.sum(zg[i] * zg[j], axis=0, keepdims=True)
    rows = []
    for i in range(K):
        for j in range(K):
            gij = gpair[(i, j)] if i <= j else gpair[(j, i)]
            rows.append(z2k[i] + z2k[j] - 2.0 * gij)
    d2z = jnp.maximum(jnp.concatenate(rows, axis=0), 0.0)
    ks = VARIANCE * jnp.exp(-0.5 * d2z / (LENGTHSCALE ** 2))
    r_kk1 = lax.broadcasted_iota(jnp.int32, (K, K, 1), 0)
    c_kk1 = lax.broadcasted_iota(jnp.int32, (K, K, 1), 1)
    b = ks.reshape(K, K, Q) + 2.0 * JITTER * (r_kk1 == c_kk1).astype(_F32)

    binv = (r_kk1 == c_kk1).astype(_F32) * jnp.ones((K, K, Q), _F32)
    aw = b
    for k in range(K):
        piv = aw[k:k + 1, k:k + 1, :]
        pr_a = aw[k:k + 1, :, :] / piv
        pr_i = binv[k:k + 1, :, :] / piv
        colf = aw[:, k:k + 1, :]
        isrow = r_kk1[:, 0:1, :] == k
        f = jnp.where(isrow, 0.0, colf)
        aw = jnp.where(isrow, pr_a, aw - f * pr_a)
        binv = jnp.where(isrow, pr_i, binv - f * pr_i)

    w = jnp.sum(kxz_t[:, None, :] * binv, axis=0)

    wg = ph[0] * w[0:1, :]
    for j in range(1, K):
        wg = wg + ph[j] * w[j:j + 1, :]

    swg = lax.dot_general(s_ref[:], wg, (((1,), (0,)), ((), ())),
                          preferred_element_type=_F32)
    quad_s = jnp.sum(wg * swg, axis=0, keepdims=True)
    mean = jnp.sum(wg * mu_ref[:], axis=0, keepdims=True)
    quad_k = (jnp.sum(w * kxz_t, axis=0, keepdims=True)
              - JITTER * jnp.sum(w * w, axis=0, keepdims=True))
    cov = VARIANCE - quad_k + quad_s
    mean_ref[:] = mean
    scale_ref[:] = jnp.sqrt(jnp.clip(cov, 0.05, None))


def kernel(X, Z, Lu_raw, mu):
    lu, l_mat, s_mat, zt, z2 = pl.pallas_call(
        _prep_body,
        out_shape=(
            jax.ShapeDtypeStruct((M, M), _F32),
            jax.ShapeDtypeStruct((M, M), _F32),
            jax.ShapeDtypeStruct((M, M), _F32),
            jax.ShapeDtypeStruct((DIM, M), _F32),
            jax.ShapeDtypeStruct((1, M), _F32),
        ),
        scratch_shapes=[pltpu.VMEM((M, M), _F32),
                        pltpu.VMEM((NB, M), _F32),
                        pltpu.VMEM((NB, NB), _F32)],
    )(Z, Lu_raw)

    mu2 = mu.reshape(M, 1)
    grid = (N // Q,)
    mean, scale = pl.pallas_call(
        _main_body,
        grid=grid,
        in_specs=[
            pl.BlockSpec((Q, DIM), lambda i: (i, 0)),
            pl.BlockSpec((DIM, M), lambda i: (0, 0)),
            pl.BlockSpec((1, M), lambda i: (0, 0)),
            pl.BlockSpec((M, M), lambda i: (0, 0)),
            pl.BlockSpec((M, 1), lambda i: (0, 0)),
        ],
        out_specs=[
            pl.BlockSpec((1, Q), lambda i: (0, i)),
            pl.BlockSpec((1, Q), lambda i: (0, i)),
        ],
        out_shape=(
            jax.ShapeDtypeStruct((1, N), _F32),
            jax.ShapeDtypeStruct((1, N), _F32),
        ),
    )(X, zt, z2, s_mat, mu2)

    return mean, scale, mu, lu, l_mat
